# baseline (speedup 1.0000x reference)
import functools

import jax
import jax.numpy as jnp
from jax import lax
from jax.experimental import pallas as pl
from jax.experimental import xla_metadata
from jax.experimental.pallas import tpu as pltpu
from jax.experimental.pallas import tpu_sc as plsc

B = 4
S = 8192
D = 1024
NC = 2
NS = 16
NW = NC * NS
SEQ_PER_W = S // NW
CH = 16
NCHUNK = SEQ_PER_W // CH
LANES = 16
VECS = (CH * D) // LANES
SC_B = 3


def _sc_kernel(x_hbm, pe_hbm, out_hbm, xbuf, pebuf, xsem, psem, osem):
    wid = lax.axis_index("s") * NC + lax.axis_index("c")
    s0 = wid * SEQ_PER_W

    def pe_load(c, slot):
        pltpu.async_copy(
            pe_hbm.at[pl.ds(s0 + c * CH, CH)], pebuf.at[slot], psem.at[slot])

    def pe_wait(c, slot):
        pltpu.make_async_copy(
            pe_hbm.at[pl.ds(s0 + c * CH, CH)], pebuf.at[slot],
            psem.at[slot]).wait()

    def x_load(c, slot):
        row = SC_B * S + s0 + c * CH
        pltpu.async_copy(
            x_hbm.at[pl.ds(row, CH)], xbuf.at[slot], xsem.at[slot])

    def x_wait(c, slot):
        row = SC_B * S + s0 + c * CH
        pltpu.make_async_copy(
            x_hbm.at[pl.ds(row, CH)], xbuf.at[slot], xsem.at[slot]).wait()

    def out_row(c):
        return s0 + c * CH

    def ostore_wait(c, slot):
        pltpu.make_async_copy(
            xbuf.at[slot], out_hbm.at[pl.ds(out_row(c), CH)],
            osem.at[slot]).wait()

    def step(t, k, skip_store_wait=False):
        x_wait(t, k)
        pe_wait(t, k % 2)
        xb = xbuf.at[k]
        pb = pebuf.at[k % 2]

        @plsc.parallel_loop(0, VECS, unroll=8)
        def _(j):
            r = lax.shift_right_logical(j, 6)
            o = pl.multiple_of(
                lax.shift_left(lax.bitwise_and(j, 63), 4), LANES)
            plsc.addupdate(
                xb.at[r].at[pl.ds(o, LANES)], pb.at[r][pl.ds(o, LANES)])

        pltpu.async_copy(
            xbuf.at[k], out_hbm.at[pl.ds(out_row(t), CH)], osem.at[k])

        if not skip_store_wait:
            ostore_wait(t, (k + 3) % 4)

        @pl.when(t + 3 < NCHUNK)
        def _():
            x_load(t + 3, (k + 3) % 4)

        @pl.when(t + 2 < NCHUNK)
        def _():
            pe_load(t + 2, k % 2)

    pe_load(0, 0)
    pe_load(1, 1)
    for k in range(3):
        x_load(k, k)
    step(0, 0, skip_store_wait=True)
    for k in range(1, 4):
        step(k, k)

    def body(g, carry):
        for k in range(4):
            step(4 * g + k, k)
        return carry

    lax.fori_loop(1, NCHUNK // 4, body, None)

    ostore_wait(NCHUNK - 1, (NCHUNK - 1) % 4)


def _tc_body(x_ref, pe_ref, o_ref):
    o_ref[...] = x_ref[...] + pe_ref[...]


def kernel(x, pe):
    x2 = x.reshape(B * S, D)
    sc_run = functools.partial(
        pl.kernel,
        mesh=plsc.VectorSubcoreMesh(core_axis_name="c", subcore_axis_name="s"),
        out_type=jax.ShapeDtypeStruct((S, D), jnp.float32),
        scratch_types=[
            pltpu.VMEM((4, CH, D), jnp.float32),
            pltpu.VMEM((2, CH, D), jnp.float32),
            pltpu.SemaphoreType.DMA((4,)),
            pltpu.SemaphoreType.DMA((2,)),
            pltpu.SemaphoreType.DMA((4,)),
        ],
    )(_sc_kernel)

    S_BLK = 2048
    ns = S // S_BLK
    with xla_metadata.set_xla_metadata(_scheduling_group_id="0"):
        sc_out = sc_run(x2, pe)
        tc_out = pl.pallas_call(
            _tc_body,
            grid=(ns, SC_B),
            in_specs=[
                pl.BlockSpec((1, S_BLK, D), lambda s, b: (b, s, 0)),
                pl.BlockSpec((S_BLK, D), lambda s, b: (s, 0)),
            ],
            out_specs=pl.BlockSpec((1, S_BLK, D), lambda s, b: (b, s, 0)),
            out_shape=jax.ShapeDtypeStruct((SC_B, S, D), x.dtype),
        )(x, pe)

    return jnp.concatenate(
        [tc_out, sc_out.reshape(1, S, D)], axis=0)


# speedup vs baseline: 1.6510x; 1.6510x over previous
import functools

import jax
import jax.numpy as jnp
from jax import lax
from jax.experimental import pallas as pl
from jax.experimental.pallas import tpu as pltpu
from jax.experimental.pallas import tpu_sc as plsc

B = 4
S = 8192
D = 1024
NC = 2
NS = 16
NW = NC * NS
SEQ_PER_W = S // NW
CH = 16
NCHUNK = SEQ_PER_W // CH
NSTEP = NCHUNK * B
LANES = 16
VECS = (CH * D) // LANES


def _sc_kernel(x_hbm, pe_hbm, out_hbm, xbuf, pebuf, xsem, psem, osem):
    wid = lax.axis_index("s") * NC + lax.axis_index("c")
    s0 = wid * SEQ_PER_W

    def row_of(c, b):
        return b * S + s0 + c * CH

    def x_load(c, b, k):
        pltpu.async_copy(
            x_hbm.at[pl.ds(row_of(c, b), CH)], xbuf.at[k], xsem.at[k])

    def x_wait(c, b, k):
        pltpu.make_async_copy(
            x_hbm.at[pl.ds(row_of(c, b), CH)], xbuf.at[k], xsem.at[k]).wait()

    def pe_load(c, p):
        pltpu.async_copy(
            pe_hbm.at[pl.ds(s0 + c * CH, CH)], pebuf.at[p], psem.at[p])

    def pe_wait(c, p):
        pltpu.make_async_copy(
            pe_hbm.at[pl.ds(s0 + c * CH, CH)], pebuf.at[p],
            psem.at[p]).wait()

    def o_store(c, b, k):
        pltpu.async_copy(
            xbuf.at[k], out_hbm.at[pl.ds(row_of(c, b), CH)], osem.at[k])

    def o_wait(c, b, k):
        pltpu.make_async_copy(
            xbuf.at[k], out_hbm.at[pl.ds(row_of(c, b), CH)],
            osem.at[k]).wait()

    def add(k, p):
        xb = xbuf.at[k]
        pb = pebuf.at[p]

        @plsc.parallel_loop(0, VECS, unroll=8)
        def _(j):
            r = lax.shift_right_logical(j, 6)
            o = pl.multiple_of(
                lax.shift_left(lax.bitwise_and(j, 63), 4), LANES)
            plsc.addupdate(
                xb.at[r].at[pl.ds(o, LANES)], pb.at[r][pl.ds(o, LANES)])

    def step(g, k, first_group):
        b = k % 4
        kk = k % 4
        p = (k // 4) % 2
        c = (2 * g + k // 4) if not first_group else (k // 4)

        x_wait(c, b, kk)
        if b == 0:
            pe_wait(c, p)
        add(kk, p)
        o_store(c, b, kk)
        if b == 3:
            if first_group:
                pe_load(c + 2, p)
            else:
                @pl.when(c + 2 < NCHUNK)
                def _():
                    pe_load(c + 2, p)
        if not (first_group and k == 0):
            if k == 0:
                c1, b1 = 2 * g - 1, 3
            else:
                c1, b1 = (2 * g + (k - 1) // 4) if not first_group \
                    else ((k - 1) // 4), (k - 1) % 4
            o_wait(c1, b1, (k - 1) % 4)
        q, r = divmod(k + 3, 8)
        b3 = r % 4
        if first_group:
            c3 = 2 * q + r // 4
            x_load(c3, b3, r % 4)
        else:
            c3 = 2 * (g + q) + r // 4

            @pl.when(c3 < NCHUNK)
            def _():
                x_load(c3, b3, r % 4)

    pe_load(0, 0)
    pe_load(1, 1)
    x_load(0, 0, 0)
    x_load(0, 1, 1)
    x_load(0, 2, 2)

    for k in range(8):
        step(0, k, first_group=True)

    def body(g, carry):
        for k in range(8):
            step(g, k, first_group=False)
        return carry

    lax.fori_loop(1, NSTEP // 8, body, None)

    o_wait(NCHUNK - 1, 3, 3)


def kernel(x, pe):
    x2 = x.reshape(B * S, D)
    run = functools.partial(
        pl.kernel,
        mesh=plsc.VectorSubcoreMesh(core_axis_name="c", subcore_axis_name="s"),
        out_type=jax.ShapeDtypeStruct((B * S, D), jnp.float32),
        scratch_types=[
            pltpu.VMEM((4, CH, D), jnp.float32),
            pltpu.VMEM((2, CH, D), jnp.float32),
            pltpu.SemaphoreType.DMA((4,)),
            pltpu.SemaphoreType.DMA((2,)),
            pltpu.SemaphoreType.DMA((4,)),
        ],
    )(_sc_kernel)
    out = run(x2, pe)
    return out.reshape(B, S, D)


# speedup vs baseline: 1.7532x; 1.0619x over previous
import functools

import jax
import jax.numpy as jnp
from jax import lax
from jax.experimental import pallas as pl
from jax.experimental.pallas import tpu as pltpu
from jax.experimental.pallas import tpu_sc as plsc

B = 4
S = 8192
D = 1024
LANES = 16

SC_SEQ0 = 6144
SC_SEQ = S - SC_SEQ0

NC = 2
NSUB = 16
NW = NC * NSUB
SEQ_PER_W = SC_SEQ // NW
CH = 16
NCHUNK = SEQ_PER_W // CH
NSTEP_SC = NCHUNK * B
VECS = (CH * D) // LANES

R = 2048
NS_BLK = SC_SEQ0 // R
NSTEP_TC = NS_BLK * B
XSLOTS = 4
PSLOTS = 2


def _sc_kernel(x_hbm, pe_hbm, out_hbm, xbuf, pebuf, xsem, psem, osem):
    wid = lax.axis_index("s") * NC + lax.axis_index("c")
    s0 = SC_SEQ0 + wid * SEQ_PER_W

    def row_of(c, b):
        return b * S + s0 + c * CH

    def x_load(c, b, k):
        pltpu.async_copy(
            x_hbm.at[pl.ds(row_of(c, b), CH)], xbuf.at[k], xsem.at[k])

    def x_wait(c, b, k):
        pltpu.make_async_copy(
            x_hbm.at[pl.ds(row_of(c, b), CH)], xbuf.at[k], xsem.at[k]).wait()

    def pe_load(c, p):
        pltpu.async_copy(
            pe_hbm.at[pl.ds(s0 + c * CH, CH)], pebuf.at[p], psem.at[p])

    def pe_wait(c, p):
        pltpu.make_async_copy(
            pe_hbm.at[pl.ds(s0 + c * CH, CH)], pebuf.at[p],
            psem.at[p]).wait()

    def o_store(c, b, k):
        pltpu.async_copy(
            xbuf.at[k], out_hbm.at[pl.ds(row_of(c, b), CH)], osem.at[k])

    def o_wait(c, b, k):
        pltpu.make_async_copy(
            xbuf.at[k], out_hbm.at[pl.ds(row_of(c, b), CH)],
            osem.at[k]).wait()

    def add(k, p):
        xb = xbuf.at[k]
        pb = pebuf.at[p]

        @plsc.parallel_loop(0, VECS, unroll=8)
        def _(j):
            r = lax.shift_right_logical(j, 6)
            o = pl.multiple_of(
                lax.shift_left(lax.bitwise_and(j, 63), 4), LANES)
            plsc.addupdate(
                xb.at[r].at[pl.ds(o, LANES)], pb.at[r][pl.ds(o, LANES)])

    pe_load(0, 0)
    pe_load(1, 1)
    for t in range(3):
        x_load(t // 4, t % 4, t % 4)

    for t in range(NSTEP_SC):
        c, b = divmod(t, 4)
        k = t % 4
        p = c % 2
        x_wait(c, b, k)
        if b == 0:
            pe_wait(c, p)
        add(k, p)
        o_store(c, b, k)
        if b == 3 and c + 2 < NCHUNK:
            pe_load(c + 2, p)
        if t >= 1:
            c1, b1 = divmod(t - 1, 4)
            o_wait(c1, b1, (t - 1) % 4)
        if t + 3 < NSTEP_SC:
            c3, b3 = divmod(t + 3, 4)
            x_load(c3, b3, (t + 3) % 4)
    o_wait(NCHUNK - 1, 3, (NSTEP_SC - 1) % 4)


def _tc_body(sc_hbm, x_hbm, pe_hbm, out_hbm, xb, peb, xsem, psem, osem):
    del sc_hbm

    def x_rows(t):
        s_blk, b = divmod(t, B)
        return b * S + s_blk * R

    def x_load(t):
        k = t % XSLOTS
        pltpu.make_async_copy(
            x_hbm.at[pl.ds(x_rows(t), R)], xb.at[k], xsem.at[k]).start()

    def x_wait(t):
        k = t % XSLOTS
        pltpu.make_async_copy(
            x_hbm.at[pl.ds(x_rows(t), R)], xb.at[k], xsem.at[k]).wait()

    def pe_load(s_blk):
        p = s_blk % PSLOTS
        pltpu.make_async_copy(
            pe_hbm.at[pl.ds(s_blk * R, R)], peb.at[p], psem.at[p]).start()

    def pe_wait(s_blk):
        p = s_blk % PSLOTS
        pltpu.make_async_copy(
            pe_hbm.at[pl.ds(s_blk * R, R)], peb.at[p], psem.at[p]).wait()

    def o_store(t):
        k = t % XSLOTS
        pltpu.make_async_copy(
            xb.at[k], out_hbm.at[pl.ds(x_rows(t), R)], osem.at[k]).start()

    def o_wait(t):
        k = t % XSLOTS
        pltpu.make_async_copy(
            xb.at[k], out_hbm.at[pl.ds(x_rows(t), R)], osem.at[k]).wait()

    pe_load(0)
    pe_load(1)
    for t in range(3):
        x_load(t)

    for t in range(NSTEP_TC):
        s_blk, b = divmod(t, B)
        x_wait(t)
        if b == 0:
            pe_wait(s_blk)
        k = t % XSLOTS
        xb[k] = xb[k] + peb[s_blk % PSLOTS]
        o_store(t)
        if b == B - 1 and s_blk + 2 < NS_BLK:
            pe_load(s_blk + 2)
        if t >= 1:
            o_wait(t - 1)
        if t + 3 < NSTEP_TC:
            x_load(t + 3)
    o_wait(NSTEP_TC - 1)


def kernel(x, pe):
    x2 = x.reshape(B * S, D)

    sc_run = functools.partial(
        pl.kernel,
        mesh=plsc.VectorSubcoreMesh(core_axis_name="c", subcore_axis_name="s"),
        out_type=jax.ShapeDtypeStruct((B * S, D), jnp.float32),
        scratch_types=[
            pltpu.VMEM((4, CH, D), jnp.float32),
            pltpu.VMEM((2, CH, D), jnp.float32),
            pltpu.SemaphoreType.DMA((4,)),
            pltpu.SemaphoreType.DMA((2,)),
            pltpu.SemaphoreType.DMA((4,)),
        ],
    )(_sc_kernel)
    partial_out = sc_run(x2, pe)

    out = pl.pallas_call(
        _tc_body,
        in_specs=[
            pl.BlockSpec(memory_space=pl.ANY),
            pl.BlockSpec(memory_space=pl.ANY),
            pl.BlockSpec(memory_space=pl.ANY),
        ],
        out_specs=pl.BlockSpec(memory_space=pl.ANY),
        out_shape=jax.ShapeDtypeStruct((B * S, D), x.dtype),
        input_output_aliases={0: 0},
        scratch_shapes=[
            pltpu.VMEM((XSLOTS, R, D), jnp.float32),
            pltpu.VMEM((PSLOTS, R, D), jnp.float32),
            pltpu.SemaphoreType.DMA((XSLOTS,)),
            pltpu.SemaphoreType.DMA((PSLOTS,)),
            pltpu.SemaphoreType.DMA((XSLOTS,)),
        ],
    )(partial_out, x2, pe)
    return out.reshape(B, S, D)


# speedup vs baseline: 1.7869x; 1.0192x over previous
import functools

import jax
import jax.numpy as jnp
from jax import lax
from jax.experimental import pallas as pl
from jax.experimental.pallas import tpu as pltpu
from jax.experimental.pallas import tpu_sc as plsc

B = 4
S = 8192
D = 1024
LANES = 16

SC_SEQ0 = 7168
SC_SEQ = S - SC_SEQ0

NC = 2
NSUB = 16
NW = NC * NSUB
SEQ_PER_W = SC_SEQ // NW
CH = 16
NCHUNK = SEQ_PER_W // CH
NSTEP_SC = NCHUNK * B
VECS = (CH * D) // LANES

R = 1024
NS_BLK = SC_SEQ0 // R
NSTEP_TC = NS_BLK * B
XSLOTS = 4
PSLOTS = 2


def _sc_kernel(x_hbm, pe_hbm, out_hbm, xbuf, pebuf, xsem, psem, osem):
    wid = lax.axis_index("s") * NC + lax.axis_index("c")
    s0 = SC_SEQ0 + wid * SEQ_PER_W

    def row_of(c, b):
        return b * S + s0 + c * CH

    def x_load(c, b, k):
        pltpu.async_copy(
            x_hbm.at[pl.ds(row_of(c, b), CH)], xbuf.at[k], xsem.at[k])

    def x_wait(c, b, k):
        pltpu.make_async_copy(
            x_hbm.at[pl.ds(row_of(c, b), CH)], xbuf.at[k], xsem.at[k]).wait()

    def pe_load(c, p):
        pltpu.async_copy(
            pe_hbm.at[pl.ds(s0 + c * CH, CH)], pebuf.at[p], psem.at[p])

    def pe_wait(c, p):
        pltpu.make_async_copy(
            pe_hbm.at[pl.ds(s0 + c * CH, CH)], pebuf.at[p],
            psem.at[p]).wait()

    def o_store(c, b, k):
        pltpu.async_copy(
            xbuf.at[k], out_hbm.at[pl.ds(row_of(c, b), CH)], osem.at[k])

    def o_wait(c, b, k):
        pltpu.make_async_copy(
            xbuf.at[k], out_hbm.at[pl.ds(row_of(c, b), CH)],
            osem.at[k]).wait()

    def add(k, p):
        xb = xbuf.at[k]
        pb = pebuf.at[p]

        @plsc.parallel_loop(0, VECS, unroll=8)
        def _(j):
            r = lax.shift_right_logical(j, 6)
            o = pl.multiple_of(
                lax.shift_left(lax.bitwise_and(j, 63), 4), LANES)
            plsc.addupdate(
                xb.at[r].at[pl.ds(o, LANES)], pb.at[r][pl.ds(o, LANES)])

    pe_load(0, 0)
    pe_load(1, 1)
    for t in range(3):
        x_load(t // 4, t % 4, t % 4)

    for t in range(NSTEP_SC):
        c, b = divmod(t, 4)
        k = t % 4
        p = c % 2
        x_wait(c, b, k)
        if b == 0:
            pe_wait(c, p)
        add(k, p)
        o_store(c, b, k)
        if b == 3 and c + 2 < NCHUNK:
            pe_load(c + 2, p)
        if t >= 1:
            c1, b1 = divmod(t - 1, 4)
            o_wait(c1, b1, (t - 1) % 4)
        if t + 3 < NSTEP_SC:
            c3, b3 = divmod(t + 3, 4)
            x_load(c3, b3, (t + 3) % 4)
    o_wait(NCHUNK - 1, 3, (NSTEP_SC - 1) % 4)


def _tc_body(sc_hbm, x_hbm, pe_hbm, out_hbm, xb, peb, xsem, psem, osem):
    del sc_hbm

    def x_rows(t):
        s_blk, b = divmod(t, B)
        return b * S + s_blk * R

    def x_load(t):
        k = t % XSLOTS
        pltpu.make_async_copy(
            x_hbm.at[pl.ds(x_rows(t), R)], xb.at[k], xsem.at[k]).start()

    def x_wait(t):
        k = t % XSLOTS
        pltpu.make_async_copy(
            x_hbm.at[pl.ds(x_rows(t), R)], xb.at[k], xsem.at[k]).wait()

    def pe_load(s_blk):
        p = s_blk % PSLOTS
        pltpu.make_async_copy(
            pe_hbm.at[pl.ds(s_blk * R, R)], peb.at[p], psem.at[p]).start()

    def pe_wait(s_blk):
        p = s_blk % PSLOTS
        pltpu.make_async_copy(
            pe_hbm.at[pl.ds(s_blk * R, R)], peb.at[p], psem.at[p]).wait()

    def o_store(t):
        k = t % XSLOTS
        pltpu.make_async_copy(
            xb.at[k], out_hbm.at[pl.ds(x_rows(t), R)], osem.at[k]).start()

    def o_wait(t):
        k = t % XSLOTS
        pltpu.make_async_copy(
            xb.at[k], out_hbm.at[pl.ds(x_rows(t), R)], osem.at[k]).wait()

    pe_load(0)
    pe_load(1)
    for t in range(3):
        x_load(t)

    for t in range(NSTEP_TC):
        s_blk, b = divmod(t, B)
        x_wait(t)
        if b == 0:
            pe_wait(s_blk)
        k = t % XSLOTS
        xb[k] = xb[k] + peb[s_blk % PSLOTS]
        o_store(t)
        if b == B - 1 and s_blk + 2 < NS_BLK:
            pe_load(s_blk + 2)
        if t >= 1:
            o_wait(t - 1)
        if t + 3 < NSTEP_TC:
            x_load(t + 3)
    o_wait(NSTEP_TC - 1)


def kernel(x, pe):
    x2 = x.reshape(B * S, D)

    sc_run = functools.partial(
        pl.kernel,
        mesh=plsc.VectorSubcoreMesh(core_axis_name="c", subcore_axis_name="s"),
        out_type=jax.ShapeDtypeStruct((B * S, D), jnp.float32),
        scratch_types=[
            pltpu.VMEM((4, CH, D), jnp.float32),
            pltpu.VMEM((2, CH, D), jnp.float32),
            pltpu.SemaphoreType.DMA((4,)),
            pltpu.SemaphoreType.DMA((2,)),
            pltpu.SemaphoreType.DMA((4,)),
        ],
    )(_sc_kernel)
    partial_out = sc_run(x2, pe)

    out = pl.pallas_call(
        _tc_body,
        in_specs=[
            pl.BlockSpec(memory_space=pl.ANY),
            pl.BlockSpec(memory_space=pl.ANY),
            pl.BlockSpec(memory_space=pl.ANY),
        ],
        out_specs=pl.BlockSpec(memory_space=pl.ANY),
        out_shape=jax.ShapeDtypeStruct((B * S, D), x.dtype),
        input_output_aliases={0: 0},
        scratch_shapes=[
            pltpu.VMEM((XSLOTS, R, D), jnp.float32),
            pltpu.VMEM((PSLOTS, R, D), jnp.float32),
            pltpu.SemaphoreType.DMA((XSLOTS,)),
            pltpu.SemaphoreType.DMA((PSLOTS,)),
            pltpu.SemaphoreType.DMA((XSLOTS,)),
        ],
    )(partial_out, x2, pe)
    return out.reshape(B, S, D)
